# baseline (speedup 1.0000x reference)
import functools

import jax
import jax.numpy as jnp
from jax import lax
from jax.experimental import pallas as pl
from jax.experimental.pallas import tpu as pltpu
from jax.experimental.pallas import tpu_sc as plsc

_T = 8192
_D = 4096
_E = 64
_TOP_K = 8
_BT = 1024

_NC = 2
_NS = 16
_NW = _NC * _NS
_ROWS_PER_W = _T // _NW
_L = 16
_NG = _ROWS_PER_W // _L


def _matmul_body(x_ref, w_ref, out_ref):
    logits = jnp.dot(x_ref[...], w_ref[...],
                     preferred_element_type=jnp.float32)
    out_ref[...] = logits.T


def _tc_logits_t(x_TD, kernel_DE):
    return pl.pallas_call(
        _matmul_body,
        grid=(_T // _BT,),
        in_specs=[
            pl.BlockSpec((_BT, _D), lambda i: (i, 0)),
            pl.BlockSpec((_D, _E), lambda i: (0, 0)),
        ],
        out_specs=pl.BlockSpec((_E, _BT), lambda i: (0, i)),
        out_shape=jax.ShapeDtypeStruct((_E, _T), jnp.float32),
        compiler_params=pltpu.CompilerParams(
            dimension_semantics=("parallel",),
        ),
    )(x_TD, kernel_DE)


def _sc_topk(logits_t):
    mesh = plsc.VectorSubcoreMesh(core_axis_name="c", subcore_axis_name="s")

    @functools.partial(
        pl.kernel,
        mesh=mesh,
        out_type=[
            jax.ShapeDtypeStruct((_TOP_K, _T), jnp.float32),
            jax.ShapeDtypeStruct((_TOP_K, _T), jnp.int32),
        ],
        scratch_types=[
            pltpu.VMEM((_E, _ROWS_PER_W), jnp.float32),
            pltpu.VMEM((_TOP_K, _ROWS_PER_W), jnp.float32),
            pltpu.VMEM((_TOP_K, _ROWS_PER_W), jnp.int32),
        ],
    )
    def topk_kernel(lt_hbm, wout_hbm, iout_hbm, lt_v, wbuf, ibuf):
        wid = lax.axis_index("s") * _NC + lax.axis_index("c")
        base = wid * _ROWS_PER_W
        pltpu.sync_copy(lt_hbm.at[:, pl.ds(base, _ROWS_PER_W)], lt_v)

        neg_inf = jnp.full((_L,), -jnp.inf, jnp.float32)
        zero_i = jnp.zeros((_L,), jnp.int32)

        def group(g, _):
            c0 = g * _L
            vs = [neg_inf] * _TOP_K
            ids = [zero_i] * _TOP_K
            for j in range(_E):
                x = lt_v[j, pl.ds(c0, _L)]
                xi = jnp.full((_L,), j, jnp.int32)
                for p in range(_TOP_K):
                    m = x > vs[p]
                    nv = jnp.where(m, x, vs[p])
                    ni = jnp.where(m, xi, ids[p])
                    x = jnp.where(m, vs[p], x)
                    xi = jnp.where(m, ids[p], xi)
                    vs[p] = nv
                    ids[p] = ni
            es = [jnp.exp(v - vs[0]) for v in vs]
            tot = es[0]
            for e in es[1:]:
                tot = tot + e
            for p in range(_TOP_K):
                wbuf[p, pl.ds(c0, _L)] = es[p] / tot
                ibuf[p, pl.ds(c0, _L)] = ids[p]
            return 0

        lax.fori_loop(0, _NG, group, 0)

        pltpu.sync_copy(wbuf, wout_hbm.at[:, pl.ds(base, _ROWS_PER_W)])
        pltpu.sync_copy(ibuf, iout_hbm.at[:, pl.ds(base, _ROWS_PER_W)])

    return topk_kernel(logits_t)


@jax.jit
def kernel(x_TD, kernel_DE):
    x_TD = jnp.asarray(x_TD, jnp.float32)
    logits_t = _tc_logits_t(x_TD, kernel_DE)
    wout_t, iout_t = _sc_topk(logits_t)
    return wout_t.T, iout_t.T


# speedup vs baseline: 1.2352x; 1.2352x over previous
import jax
import jax.numpy as jnp
from jax.experimental import pallas as pl
from jax.experimental.pallas import tpu as pltpu

_T = 8192
_D = 4096
_E = 64
_TOP_K = 8
_BT = 1024


def _router_body(x_ref, w_ref, wout_ref, iout_ref):
    x = x_ref[...]
    w = w_ref[...]
    logits = jnp.dot(x, w, preferred_element_type=jnp.float32)

    coli = jax.lax.broadcasted_iota(jnp.int32, (_BT, _E), 1)
    bits = jax.lax.bitcast_convert_type(logits, jnp.int32)
    key_bits = (bits & -64) | (63 - coli)
    key = jax.lax.bitcast_convert_type(key_bits, jnp.float32)

    picked = []
    for _ in range(_TOP_K):
        m = jnp.max(key, axis=1, keepdims=True)
        picked.append(m)
        key = jnp.where(key == m, -jnp.inf, key)

    kcat = jnp.concatenate(picked, axis=1)
    kcat_bits = jax.lax.bitcast_convert_type(kcat, jnp.int32)
    iout_ref[...] = 63 - (kcat_bits & 63)
    v = jax.lax.bitcast_convert_type(kcat_bits & -64, jnp.float32)
    e = jnp.exp(v - v[:, 0:1])
    wout_ref[...] = e / jnp.sum(e, axis=1, keepdims=True)


@jax.jit
def kernel(x_TD, kernel_DE):
    x_TD = jnp.asarray(x_TD, jnp.float32)
    grid = (_T // _BT,)
    wout, iout = pl.pallas_call(
        _router_body,
        grid=grid,
        in_specs=[
            pl.BlockSpec((_BT, _D), lambda i: (i, 0)),
            pl.BlockSpec((_D, _E), lambda i: (0, 0)),
        ],
        out_specs=[
            pl.BlockSpec((_BT, _TOP_K), lambda i: (i, 0)),
            pl.BlockSpec((_BT, _TOP_K), lambda i: (i, 0)),
        ],
        out_shape=[
            jax.ShapeDtypeStruct((_T, _TOP_K), jnp.float32),
            jax.ShapeDtypeStruct((_T, _TOP_K), jnp.int32),
        ],
        compiler_params=pltpu.CompilerParams(
            dimension_semantics=("parallel",),
        ),
    )(x_TD, kernel_DE)
    return wout, iout
